# baseline (speedup 1.0000x reference)
import functools

import jax
import jax.numpy as jnp
from jax import lax
from jax.experimental import pallas as pl
from jax.experimental.pallas import tpu as pltpu
from jax.experimental.pallas import tpu_sc as plsc

N = 10000
E = 320000
D = 128
DE = 16
H = 42

NCORES = 2
NSUB = 16
CH = 64
NCH0 = 304
NCH1 = 16
EPAD = NSUB * (NCH0 + NCH1) * CH
NPAD = 10240
STRIPE = NPAD // NSUB

EB = 8192
NB = 1000


def _edge_w_body(eat_ref, w1t_ref, b1_ref, w2t_ref, b2_ref, o_ref):
    eh = jnp.dot(w1t_ref[...], eat_ref[...], preferred_element_type=jnp.float32)
    eh = jnp.maximum(eh + b1_ref[...], 0.0)
    o_ref[0] = jnp.dot(w2t_ref[...], eh, preferred_element_type=jnp.float32) + b2_ref[...]


def _edge_weights(ea_t, We1, be1, We2, be2):
    nblk = EPAD // EB
    return pl.pallas_call(
        _edge_w_body,
        grid=(nblk,),
        in_specs=[
            pl.BlockSpec((DE, EB), lambda i: (0, i)),
            pl.BlockSpec((H, DE), lambda i: (0, 0)),
            pl.BlockSpec((H, 1), lambda i: (0, 0)),
            pl.BlockSpec((1, H), lambda i: (0, 0)),
            pl.BlockSpec((1, 1), lambda i: (0, 0)),
        ],
        out_specs=pl.BlockSpec((1, 1, EB), lambda i: (i, 0, 0)),
        out_shape=jax.ShapeDtypeStruct((nblk, 1, EB), jnp.float32),
    )(ea_t, We1.T, be1.reshape(H, 1), We2.T, be2.reshape(1, 1))


def _node_mlp_body(x_ref, wn_ref, bn_ref, wm_ref, h_ref, m_ref):
    h = jnp.dot(x_ref[...], wn_ref[...], preferred_element_type=jnp.float32)
    h = jnp.maximum(h + bn_ref[...], 0.0)
    h_ref[...] = h
    m_ref[...] = jnp.dot(h, wm_ref[...], preferred_element_type=jnp.float32)


def _node_mlp(x, Wn1, bn1, Wm0):
    return pl.pallas_call(
        _node_mlp_body,
        grid=(N // NB,),
        in_specs=[
            pl.BlockSpec((NB, D), lambda i: (i, 0)),
            pl.BlockSpec((D, D), lambda i: (0, 0)),
            pl.BlockSpec((1, D), lambda i: (0, 0)),
            pl.BlockSpec((D, D), lambda i: (0, 0)),
        ],
        out_specs=[
            pl.BlockSpec((NB, D), lambda i: (i, 0)),
            pl.BlockSpec((NB, D), lambda i: (i, 0)),
        ],
        out_shape=[
            jax.ShapeDtypeStruct((N, D), jnp.float32),
            jax.ShapeDtypeStruct((N, D), jnp.float32),
        ],
    )(x, Wn1, bn1.reshape(1, D), Wm0)


def _gru_core(aggp_ref, h_ref, wih_ref, bih_ref, whh_ref, bhh_ref):
    agg = aggp_ref[0] + aggp_ref[1]
    h = h_ref[...]
    gi = jnp.dot(agg, wih_ref[...], preferred_element_type=jnp.float32) + bih_ref[...]
    gh = jnp.dot(h, whh_ref[...], preferred_element_type=jnp.float32) + bhh_ref[...]
    r = jax.nn.sigmoid(gi[:, :D] + gh[:, :D])
    z = jax.nn.sigmoid(gi[:, D:2 * D] + gh[:, D:2 * D])
    n = jnp.tanh(gi[:, 2 * D:] + r * gh[:, 2 * D:])
    return (1.0 - z) * n + z * h


def _gru_msg_body(aggp_ref, h_ref, wih_ref, bih_ref, whh_ref, bhh_ref, wm_ref,
                  hn_ref, m_ref):
    hn = _gru_core(aggp_ref, h_ref, wih_ref, bih_ref, whh_ref, bhh_ref)
    hn_ref[...] = hn
    m_ref[...] = jnp.dot(hn, wm_ref[...], preferred_element_type=jnp.float32)


def _gru_final_body(aggp_ref, h_ref, wih_ref, bih_ref, whh_ref, bhh_ref, o_ref):
    hn = _gru_core(aggp_ref, h_ref, wih_ref, bih_ref, whh_ref, bhh_ref)
    o_ref[...] = jnp.maximum(hn, 0.0)


def _gru_specs():
    return [
        pl.BlockSpec((2, NB, D), lambda i: (0, i, 0)),
        pl.BlockSpec((NB, D), lambda i: (i, 0)),
        pl.BlockSpec((D, 3 * D), lambda i: (0, 0)),
        pl.BlockSpec((1, 3 * D), lambda i: (0, 0)),
        pl.BlockSpec((D, 3 * D), lambda i: (0, 0)),
        pl.BlockSpec((1, 3 * D), lambda i: (0, 0)),
    ]


def _gru_msg(aggp, h, W_ih, b_ih, W_hh, b_hh, Wm):
    return pl.pallas_call(
        _gru_msg_body,
        grid=(N // NB,),
        in_specs=_gru_specs() + [pl.BlockSpec((D, D), lambda i: (0, 0))],
        out_specs=[
            pl.BlockSpec((NB, D), lambda i: (i, 0)),
            pl.BlockSpec((NB, D), lambda i: (i, 0)),
        ],
        out_shape=[
            jax.ShapeDtypeStruct((N, D), jnp.float32),
            jax.ShapeDtypeStruct((N, D), jnp.float32),
        ],
    )(aggp, h, W_ih, b_ih.reshape(1, 3 * D), W_hh, b_hh.reshape(1, 3 * D), Wm)


def _gru_final(aggp, h, W_ih, b_ih, W_hh, b_hh):
    return pl.pallas_call(
        _gru_final_body,
        grid=(N // NB,),
        in_specs=_gru_specs(),
        out_specs=pl.BlockSpec((NB, D), lambda i: (i, 0)),
        out_shape=jax.ShapeDtypeStruct((N, D), jnp.float32),
    )(aggp, h, W_ih, b_ih.reshape(1, 3 * D), W_hh, b_hh.reshape(1, 3 * D))


NROWS = 4
NIDS = 4


def _sc_body(m_hbm, src_hbm, dst_hbm, w_hbm, z_hbm, out_hbm,
             sidx, didx, wbuf, rows, idsems, gsems, ssems, agg_sh):
    c = lax.axis_index("c")
    s = lax.axis_index("s")
    pltpu.sync_copy(z_hbm, agg_sh.at[pl.ds(s * STRIPE, STRIPE)])
    plsc.subcore_barrier()
    nch = jnp.where(c == 0, NCH0, NCH1)
    base = jnp.where(c == 0, s * NCH0, NSUB * NCH0 + s * NCH1) * CH

    def start_ids(k, i):
        eb = base + k * CH
        pltpu.async_copy(src_hbm.at[pl.ds(eb, CH)], sidx[i], idsems[i])
        pltpu.async_copy(dst_hbm.at[pl.ds(eb, CH)], didx[i], idsems[i])
        pltpu.async_copy(w_hbm.at[pl.ds(eb, CH)], wbuf[i], idsems[i])

    def wait_ids(i):
        pltpu.make_async_copy(src_hbm.at[pl.ds(0, CH)], sidx[i], idsems[i]).wait()
        pltpu.make_async_copy(dst_hbm.at[pl.ds(0, CH)], didx[i], idsems[i]).wait()
        pltpu.make_async_copy(w_hbm.at[pl.ds(0, CH)], wbuf[i], idsems[i]).wait()

    def start_gather(i, b):
        pltpu.async_copy(m_hbm.at[sidx[i]], rows[b], gsems[b])

    def wait_gather(b):
        pltpu.make_async_copy(m_hbm.at[sidx[0]], rows[b], gsems[b]).wait()

    def start_scatter(i, b):
        pltpu.async_copy(rows[b], agg_sh.at[didx[i]], ssems[b], add=True)

    def wait_scatter(b):
        pltpu.make_async_copy(rows[b], agg_sh.at[didx[0]], ssems[b]).wait()

    def scale(i, b):
        rb = rows[b]
        for g in range(CH // 16):
            wvec = wbuf[i][pl.ds(g * 16, 16)]
            for l in range(16):
                wv = jnp.full((16,), wvec[l], dtype=jnp.float32)
                e = g * 16 + l
                for j in range(D // 16):
                    rb[e, pl.ds(j * 16, 16)] = rb[e, pl.ds(j * 16, 16)] * wv

    start_ids(0, 0)
    start_ids(1, 1)
    wait_ids(0)
    start_gather(0, 0)
    wait_ids(1)
    start_gather(1, 1)

    def quad(t, carry):
        for u in range(NIDS):
            k = NIDS * t + u
            b = u % NROWS
            wait_gather(b)
            bn = (u + 2) % NROWS
            if u < 2:
                @pl.when(t > 0)
                def _():
                    wait_scatter(bn)
            else:
                wait_scatter(bn)

            @pl.when(k + 2 < nch)
            def _():
                start_ids(k + 2, bn)

            scale(u, b)
            start_scatter(u, b)

            @pl.when(k + 2 < nch)
            def _():
                wait_ids(bn)
                start_gather(bn, bn)
        return carry

    lax.fori_loop(0, nch // NIDS, quad, 0)
    wait_scatter(2)
    wait_scatter(3)
    plsc.subcore_barrier()
    pltpu.sync_copy(agg_sh.at[pl.ds(s * STRIPE, STRIPE)],
                    out_hbm.at[c, pl.ds(s * STRIPE, STRIPE)])


def _sc_scatter(m, src, dst, w, zeros_sc):
    mesh = plsc.VectorSubcoreMesh(core_axis_name="c", subcore_axis_name="s")
    k = functools.partial(
        pl.kernel,
        out_type=jax.ShapeDtypeStruct((NCORES, NPAD, D), jnp.float32),
        mesh=mesh,
        scratch_types=[
            [pltpu.VMEM((CH,), jnp.int32) for _ in range(NIDS)],
            [pltpu.VMEM((CH,), jnp.int32) for _ in range(NIDS)],
            [pltpu.VMEM((CH,), jnp.float32) for _ in range(NIDS)],
            [pltpu.VMEM((CH, D), jnp.float32) for _ in range(NROWS)],
            [pltpu.SemaphoreType.DMA for _ in range(NIDS)],
            [pltpu.SemaphoreType.DMA for _ in range(NROWS)],
            [pltpu.SemaphoreType.DMA for _ in range(NROWS)],
            pltpu.VMEM_SHARED((NPAD, D), jnp.float32),
        ],
    )(_sc_body)
    return k(m, src, dst, w, zeros_sc)


def kernel(x, edge_index, edge_attr, We1, be1, We2, be2, Wn1, bn1, W_msg,
           W_ih, b_ih, W_hh, b_hh):
    pad = EPAD - E
    src = jnp.concatenate(
        [edge_index[0].astype(jnp.int32), jnp.zeros((pad,), jnp.int32)])
    dst = jnp.concatenate(
        [edge_index[1].astype(jnp.int32), jnp.zeros((pad,), jnp.int32)])
    ea_t = jnp.concatenate(
        [edge_attr, jnp.zeros((pad, DE), jnp.float32)], axis=0).T

    w2 = _edge_weights(ea_t, We1, be1, We2, be2)
    w_flat = w2.reshape(EPAD)
    w_flat = jnp.where(jnp.arange(EPAD) < E, w_flat, 0.0)

    zeros_sc = jnp.zeros((STRIPE, D), jnp.float32)

    h, m = _node_mlp(x, Wn1, bn1, W_msg[0])
    aggp = _sc_scatter(m, src, dst, w_flat, zeros_sc)
    h, m = _gru_msg(aggp, h, W_ih, b_ih, W_hh, b_hh, W_msg[1])
    aggp = _sc_scatter(m, src, dst, w_flat, zeros_sc)
    return _gru_final(aggp, h, W_ih, b_ih, W_hh, b_hh)


# speedup vs baseline: 1.1549x; 1.1549x over previous
import functools

import jax
import jax.numpy as jnp
from jax import lax
from jax.experimental import pallas as pl
from jax.experimental.pallas import tpu as pltpu
from jax.experimental.pallas import tpu_sc as plsc

N = 10000
E = 320000
D = 128
DE = 16
H = 42

NCORES = 2
NSUB = 16
CH = 64
NCH0 = 304
NCH1 = 16
EPAD = NSUB * (NCH0 + NCH1) * CH
NPAD = 10240
STRIPE = NPAD // NSUB

EB = 8192
NB = 1000


def _edge_w_body(eat_ref, w1t_ref, b1_ref, w2t_ref, b2_ref, o_ref):
    eh = jnp.dot(w1t_ref[...], eat_ref[...], preferred_element_type=jnp.float32)
    eh = jnp.maximum(eh + b1_ref[...], 0.0)
    o_ref[0] = jnp.dot(w2t_ref[...], eh, preferred_element_type=jnp.float32) + b2_ref[...]


def _edge_weights(ea_t, We1, be1, We2, be2):
    nblk = EPAD // EB
    return pl.pallas_call(
        _edge_w_body,
        grid=(nblk,),
        in_specs=[
            pl.BlockSpec((DE, EB), lambda i: (0, i)),
            pl.BlockSpec((H, DE), lambda i: (0, 0)),
            pl.BlockSpec((H, 1), lambda i: (0, 0)),
            pl.BlockSpec((1, H), lambda i: (0, 0)),
            pl.BlockSpec((1, 1), lambda i: (0, 0)),
        ],
        out_specs=pl.BlockSpec((1, 1, EB), lambda i: (i, 0, 0)),
        out_shape=jax.ShapeDtypeStruct((nblk, 1, EB), jnp.float32),
    )(ea_t, We1.T, be1.reshape(H, 1), We2.T, be2.reshape(1, 1))


def _node_mlp_body(x_ref, wn_ref, bn_ref, wm_ref, h_ref, m_ref):
    h = jnp.dot(x_ref[...], wn_ref[...], preferred_element_type=jnp.float32)
    h = jnp.maximum(h + bn_ref[...], 0.0)
    h_ref[...] = h
    m_ref[...] = jnp.dot(h, wm_ref[...], preferred_element_type=jnp.float32)


def _node_mlp(x, Wn1, bn1, Wm0):
    return pl.pallas_call(
        _node_mlp_body,
        grid=(N // NB,),
        in_specs=[
            pl.BlockSpec((NB, D), lambda i: (i, 0)),
            pl.BlockSpec((D, D), lambda i: (0, 0)),
            pl.BlockSpec((1, D), lambda i: (0, 0)),
            pl.BlockSpec((D, D), lambda i: (0, 0)),
        ],
        out_specs=[
            pl.BlockSpec((NB, D), lambda i: (i, 0)),
            pl.BlockSpec((NB, D), lambda i: (i, 0)),
        ],
        out_shape=[
            jax.ShapeDtypeStruct((N, D), jnp.float32),
            jax.ShapeDtypeStruct((N, D), jnp.float32),
        ],
    )(x, Wn1, bn1.reshape(1, D), Wm0)


def _gru_core(aggp_ref, h_ref, wih_ref, bih_ref, whh_ref, bhh_ref):
    agg = aggp_ref[0] + aggp_ref[1]
    h = h_ref[...]
    gi = jnp.dot(agg, wih_ref[...], preferred_element_type=jnp.float32) + bih_ref[...]
    gh = jnp.dot(h, whh_ref[...], preferred_element_type=jnp.float32) + bhh_ref[...]
    r = jax.nn.sigmoid(gi[:, :D] + gh[:, :D])
    z = jax.nn.sigmoid(gi[:, D:2 * D] + gh[:, D:2 * D])
    n = jnp.tanh(gi[:, 2 * D:] + r * gh[:, 2 * D:])
    return (1.0 - z) * n + z * h


def _gru_msg_body(aggp_ref, h_ref, wih_ref, bih_ref, whh_ref, bhh_ref, wm_ref,
                  hn_ref, m_ref):
    hn = _gru_core(aggp_ref, h_ref, wih_ref, bih_ref, whh_ref, bhh_ref)
    hn_ref[...] = hn
    m_ref[...] = jnp.dot(hn, wm_ref[...], preferred_element_type=jnp.float32)


def _gru_final_body(aggp_ref, h_ref, wih_ref, bih_ref, whh_ref, bhh_ref, o_ref):
    hn = _gru_core(aggp_ref, h_ref, wih_ref, bih_ref, whh_ref, bhh_ref)
    o_ref[...] = jnp.maximum(hn, 0.0)


def _gru_specs():
    return [
        pl.BlockSpec((2, NB, D), lambda i: (0, i, 0)),
        pl.BlockSpec((NB, D), lambda i: (i, 0)),
        pl.BlockSpec((D, 3 * D), lambda i: (0, 0)),
        pl.BlockSpec((1, 3 * D), lambda i: (0, 0)),
        pl.BlockSpec((D, 3 * D), lambda i: (0, 0)),
        pl.BlockSpec((1, 3 * D), lambda i: (0, 0)),
    ]


def _gru_msg(aggp, h, W_ih, b_ih, W_hh, b_hh, Wm):
    return pl.pallas_call(
        _gru_msg_body,
        grid=(N // NB,),
        in_specs=_gru_specs() + [pl.BlockSpec((D, D), lambda i: (0, 0))],
        out_specs=[
            pl.BlockSpec((NB, D), lambda i: (i, 0)),
            pl.BlockSpec((NB, D), lambda i: (i, 0)),
        ],
        out_shape=[
            jax.ShapeDtypeStruct((N, D), jnp.float32),
            jax.ShapeDtypeStruct((N, D), jnp.float32),
        ],
    )(aggp, h, W_ih, b_ih.reshape(1, 3 * D), W_hh, b_hh.reshape(1, 3 * D), Wm)


def _gru_final(aggp, h, W_ih, b_ih, W_hh, b_hh):
    return pl.pallas_call(
        _gru_final_body,
        grid=(N // NB,),
        in_specs=_gru_specs(),
        out_specs=pl.BlockSpec((NB, D), lambda i: (i, 0)),
        out_shape=jax.ShapeDtypeStruct((N, D), jnp.float32),
    )(aggp, h, W_ih, b_ih.reshape(1, 3 * D), W_hh, b_hh.reshape(1, 3 * D))


NROWS = 4
NIDS = 8


def _sc_body(m_hbm, src_hbm, dst_hbm, w_hbm, z_hbm, out_hbm,
             sidx, didx, wbuf, rows, idsems, gsems, ssems, agg_sh):
    c = lax.axis_index("c")
    s = lax.axis_index("s")
    pltpu.sync_copy(z_hbm, agg_sh.at[pl.ds(s * STRIPE, STRIPE)])
    plsc.subcore_barrier()
    nch = jnp.where(c == 0, NCH0, NCH1)
    base = jnp.where(c == 0, s * NCH0, NSUB * NCH0 + s * NCH1) * CH

    def start_ids(k, i):
        eb = base + k * CH
        pltpu.async_copy(src_hbm.at[pl.ds(eb, CH)], sidx[i], idsems[i])
        pltpu.async_copy(dst_hbm.at[pl.ds(eb, CH)], didx[i], idsems[i])
        pltpu.async_copy(w_hbm.at[pl.ds(eb, CH)], wbuf[i], idsems[i])

    def wait_ids(i):
        pltpu.make_async_copy(src_hbm.at[pl.ds(0, CH)], sidx[i], idsems[i]).wait()
        pltpu.make_async_copy(dst_hbm.at[pl.ds(0, CH)], didx[i], idsems[i]).wait()
        pltpu.make_async_copy(w_hbm.at[pl.ds(0, CH)], wbuf[i], idsems[i]).wait()

    def start_gather(i, b):
        pltpu.async_copy(m_hbm.at[sidx[i]], rows[b], gsems[b])

    def wait_gather(b):
        pltpu.make_async_copy(m_hbm.at[sidx[0]], rows[b], gsems[b]).wait()

    def start_scatter(i, b):
        pltpu.async_copy(rows[b], agg_sh.at[didx[i]], ssems[b], add=True)

    def wait_scatter(b):
        pltpu.make_async_copy(rows[b], agg_sh.at[didx[0]], ssems[b]).wait()

    def scale(i, b):
        rb = rows[b]

        def grp(g, c2):
            wvec = wbuf[i][pl.ds(g * 16, 16)]
            for l in range(16):
                wv = jnp.full((16,), wvec[l], dtype=jnp.float32)
                e = g * 16 + l
                for j in range(D // 16):
                    rb[e, pl.ds(j * 16, 16)] = rb[e, pl.ds(j * 16, 16)] * wv
            return c2

        lax.fori_loop(0, CH // 16, grp, 0)

    for i in range(NROWS):
        start_ids(i, i)
    wait_ids(0)
    start_gather(0, 0)
    wait_ids(1)
    start_gather(1, 1)

    def octet(t, carry):
        for u in range(NIDS):
            k = NIDS * t + u
            b = u % NROWS
            wait_gather(b)
            scale(u, b)
            start_scatter(u, b)
            bn = (u + 2) % NROWS
            if u < 2:
                @pl.when(t > 0)
                def _():
                    wait_scatter(bn)
            else:
                wait_scatter(bn)

            @pl.when(k + 4 < nch)
            def _():
                start_ids(k + 4, (u + 4) % NIDS)

            i2 = (u + 2) % NIDS

            @pl.when(k + 2 < nch)
            def _():
                wait_ids(i2)
                start_gather(i2, bn)
        return carry

    lax.fori_loop(0, nch // NIDS, octet, 0)
    wait_scatter(2)
    wait_scatter(3)
    plsc.subcore_barrier()
    pltpu.sync_copy(agg_sh.at[pl.ds(s * STRIPE, STRIPE)],
                    out_hbm.at[c, pl.ds(s * STRIPE, STRIPE)])


def _sc_scatter(m, src, dst, w, zeros_sc):
    mesh = plsc.VectorSubcoreMesh(core_axis_name="c", subcore_axis_name="s")
    k = functools.partial(
        pl.kernel,
        out_type=jax.ShapeDtypeStruct((NCORES, NPAD, D), jnp.float32),
        mesh=mesh,
        scratch_types=[
            [pltpu.VMEM((CH,), jnp.int32) for _ in range(NIDS)],
            [pltpu.VMEM((CH,), jnp.int32) for _ in range(NIDS)],
            [pltpu.VMEM((CH,), jnp.float32) for _ in range(NIDS)],
            [pltpu.VMEM((CH, D), jnp.float32) for _ in range(NROWS)],
            [pltpu.SemaphoreType.DMA for _ in range(NIDS)],
            [pltpu.SemaphoreType.DMA for _ in range(NROWS)],
            [pltpu.SemaphoreType.DMA for _ in range(NROWS)],
            pltpu.VMEM_SHARED((NPAD, D), jnp.float32),
        ],
    )(_sc_body)
    return k(m, src, dst, w, zeros_sc)


def kernel(x, edge_index, edge_attr, We1, be1, We2, be2, Wn1, bn1, W_msg,
           W_ih, b_ih, W_hh, b_hh):
    pad = EPAD - E
    src = jnp.concatenate(
        [edge_index[0].astype(jnp.int32), jnp.zeros((pad,), jnp.int32)])
    dst = jnp.concatenate(
        [edge_index[1].astype(jnp.int32), jnp.zeros((pad,), jnp.int32)])
    ea_t = jnp.concatenate(
        [edge_attr, jnp.zeros((pad, DE), jnp.float32)], axis=0).T

    w2 = _edge_weights(ea_t, We1, be1, We2, be2)
    w_flat = w2.reshape(EPAD)
    w_flat = jnp.where(jnp.arange(EPAD) < E, w_flat, 0.0)

    zeros_sc = jnp.zeros((STRIPE, D), jnp.float32)

    h, m = _node_mlp(x, Wn1, bn1, W_msg[0])
    aggp = _sc_scatter(m, src, dst, w_flat, zeros_sc)
    h, m = _gru_msg(aggp, h, W_ih, b_ih, W_hh, b_hh, W_msg[1])
    aggp = _sc_scatter(m, src, dst, w_flat, zeros_sc)
    return _gru_final(aggp, h, W_ih, b_ih, W_hh, b_hh)
